# baseline (speedup 1.0000x reference)
import jax
import jax.numpy as jnp
from jax import lax
from jax.experimental import pallas as pl
from jax.experimental.pallas import tpu as pltpu
from jax.experimental.pallas import tpu_sc as plsc

B = 16384
EMB = 100
NFX = 56
NC, NS = 2, 16
NW = NC * NS
BPW = B // NW
CH = 128
NCH = BPW // CH
H1, H2 = 128, 32
BLK = 2048


def _gather_body(uidx_ref, aidx_ref, users_ref, animes_ref,
                 urows_out, arows_out, idx_v, urows_v, arows_v, sem):
    wid = lax.axis_index("s") * NC + lax.axis_index("c")
    base = wid * BPW
    pltpu.sync_copy(uidx_ref.at[pl.ds(wid * NCH, NCH)], idx_v.at[pl.ds(0, NCH)])
    pltpu.sync_copy(aidx_ref.at[pl.ds(wid * NCH, NCH)], idx_v.at[pl.ds(NCH, NCH)])
    copies = []
    for j in range(NCH):
        copies.append(pltpu.async_copy(
            users_ref.at[idx_v.at[j]], urows_v.at[pl.ds(j * CH, CH)], sem))
    for j in range(NCH):
        copies.append(pltpu.async_copy(
            animes_ref.at[idx_v.at[NCH + j]], arows_v.at[pl.ds(j * CH, CH)], sem))
    for c in copies:
        c.wait()
    pltpu.sync_copy(urows_v, urows_out.at[pl.ds(base, BPW)])
    pltpu.sync_copy(arows_v, arows_out.at[pl.ds(base, BPW)])


_gather = pl.kernel(
    _gather_body,
    out_type=(jax.ShapeDtypeStruct((B, EMB), jnp.float32),
              jax.ShapeDtypeStruct((B, EMB), jnp.float32)),
    mesh=plsc.VectorSubcoreMesh(core_axis_name="c", subcore_axis_name="s",
                                num_cores=NC, num_subcores=NS),
    scratch_types=[
        pltpu.VMEM((2 * NCH, CH), jnp.int32),
        pltpu.VMEM((BPW, EMB), jnp.float32),
        pltpu.VMEM((BPW, EMB), jnp.float32),
        pltpu.SemaphoreType.DMA,
    ],
    compiler_params=pltpu.CompilerParams(use_tc_tiling_on_sc=False),
)


def _mlp_body(u_ref, a_ref, x_ref, w1u_ref, w1a_ref, w1f_ref, b1_ref,
              w2_ref, b2_ref, w3_ref, b3_ref, o_ref):
    f = x_ref[:, 2:]
    h = (jnp.dot(u_ref[:], w1u_ref[:], preferred_element_type=jnp.float32)
         + jnp.dot(a_ref[:], w1a_ref[:], preferred_element_type=jnp.float32)
         + jnp.dot(f, w1f_ref[:], preferred_element_type=jnp.float32)
         + b1_ref[:])
    h = jnp.maximum(h, 0.0)
    h2 = jnp.dot(h, w2_ref[:], preferred_element_type=jnp.float32) + b2_ref[:]
    h2 = jnp.maximum(h2, 0.0)
    z = jnp.sum(h2 * w3_ref[:], axis=1, keepdims=True) + b3_ref[:]
    o_ref[:] = jax.nn.sigmoid(z)


_mlp = pl.pallas_call(
    _mlp_body,
    grid=(B // BLK,),
    in_specs=[
        pl.BlockSpec((BLK, EMB), lambda i: (i, 0)),
        pl.BlockSpec((BLK, EMB), lambda i: (i, 0)),
        pl.BlockSpec((BLK, NFX), lambda i: (i, 0)),
        pl.BlockSpec((EMB, H1), lambda i: (0, 0)),
        pl.BlockSpec((EMB, H1), lambda i: (0, 0)),
        pl.BlockSpec((NFX - 2, H1), lambda i: (0, 0)),
        pl.BlockSpec((1, H1), lambda i: (0, 0)),
        pl.BlockSpec((H1, H2), lambda i: (0, 0)),
        pl.BlockSpec((1, H2), lambda i: (0, 0)),
        pl.BlockSpec((1, H2), lambda i: (0, 0)),
        pl.BlockSpec((1, 1), lambda i: (0, 0)),
    ],
    out_specs=pl.BlockSpec((BLK, 1), lambda i: (i, 0)),
    out_shape=jax.ShapeDtypeStruct((B, 1), jnp.float32),
    compiler_params=pltpu.CompilerParams(dimension_semantics=("arbitrary",)),
)


def kernel(x, users, animes, W1, b1, W2, b2, W3, b3):
    uidx = x[:, 0].astype(jnp.int32).reshape(B // CH, CH)
    aidx = x[:, 1].astype(jnp.int32).reshape(B // CH, CH)
    urows, arows = _gather(uidx, aidx, users, animes)
    w1u = W1[:, :EMB].T
    w1a = W1[:, EMB:2 * EMB].T
    w1f = W1[:, 2 * EMB:].T
    return _mlp(urows, arows, x, w1u, w1a, w1f, b1.reshape(1, H1),
                W2.T, b2.reshape(1, H2), W3, b3.reshape(1, 1))


# speedup vs baseline: 1.3342x; 1.3342x over previous
import jax
import jax.numpy as jnp
from jax import lax
from jax.experimental import pallas as pl
from jax.experimental.pallas import tpu as pltpu
from jax.experimental.pallas import tpu_sc as plsc

B = 16384
EMB = 100
PITCH = 128
NFX = 56
NC, NS = 2, 16
NW = NC * NS
BPW = B // NW
CH = 128
NCH = BPW // CH
H1, H2 = 128, 32
BLK = 2048


def _gather_body(uidx_ref, aidx_ref, users_ref, animes_ref,
                 urows_out, arows_out, idx_v, b0, b1, b2, b3, gsem, osem):
    bufs = (b0, b1, b2, b3)
    wid = lax.axis_index("s") * NC + lax.axis_index("c")
    base = wid * BPW
    pltpu.sync_copy(uidx_ref.at[pl.ds(wid * NCH, NCH)], idx_v.at[pl.ds(0, NCH)])
    pltpu.sync_copy(aidx_ref.at[pl.ds(wid * NCH, NCH)], idx_v.at[pl.ds(NCH, NCH)])
    ug = [pltpu.async_copy(users_ref.at[idx_v.at[k]], bufs[k], gsem)
          for k in range(NCH)]
    uo = []
    for k in range(NCH):
        ug[k].wait()
        uo.append(pltpu.async_copy(
            bufs[k], urows_out.at[pl.ds(base + k * CH, CH)], osem))
    ag, ao = [], []
    for k in range(NCH):
        uo[k].wait()
        ag.append(pltpu.async_copy(
            animes_ref.at[idx_v.at[NCH + k]], bufs[k], gsem))
    for k in range(NCH):
        ag[k].wait()
        ao.append(pltpu.async_copy(
            bufs[k], arows_out.at[pl.ds(base + k * CH, CH)], osem))
    for k in range(NCH):
        ao[k].wait()


_gather = pl.kernel(
    _gather_body,
    out_type=(jax.ShapeDtypeStruct((B, PITCH), jnp.float32),
              jax.ShapeDtypeStruct((B, PITCH), jnp.float32)),
    mesh=plsc.VectorSubcoreMesh(core_axis_name="c", subcore_axis_name="s",
                                num_cores=NC, num_subcores=NS),
    scratch_types=[
        pltpu.VMEM((2 * NCH, CH), jnp.int32),
        pltpu.VMEM((CH, PITCH), jnp.float32),
        pltpu.VMEM((CH, PITCH), jnp.float32),
        pltpu.VMEM((CH, PITCH), jnp.float32),
        pltpu.VMEM((CH, PITCH), jnp.float32),
        pltpu.SemaphoreType.DMA,
        pltpu.SemaphoreType.DMA,
    ],
)


def _mlp_body(u_ref, a_ref, x_ref, w1u_ref, w1a_ref, w1f_ref, b1_ref,
              w2_ref, b2_ref, w3_ref, b3_ref, o_ref):
    f = x_ref[:, 2:]
    h = (jnp.dot(u_ref[:, :EMB], w1u_ref[:], preferred_element_type=jnp.float32)
         + jnp.dot(a_ref[:, :EMB], w1a_ref[:], preferred_element_type=jnp.float32)
         + jnp.dot(f, w1f_ref[:], preferred_element_type=jnp.float32)
         + b1_ref[:])
    h = jnp.maximum(h, 0.0)
    h2 = jnp.dot(h, w2_ref[:], preferred_element_type=jnp.float32) + b2_ref[:]
    h2 = jnp.maximum(h2, 0.0)
    z = jnp.sum(h2 * w3_ref[:], axis=1, keepdims=True) + b3_ref[:]
    o_ref[:] = jax.nn.sigmoid(z)


_mlp = pl.pallas_call(
    _mlp_body,
    grid=(B // BLK,),
    in_specs=[
        pl.BlockSpec((BLK, PITCH), lambda i: (i, 0)),
        pl.BlockSpec((BLK, PITCH), lambda i: (i, 0)),
        pl.BlockSpec((BLK, NFX), lambda i: (i, 0)),
        pl.BlockSpec((EMB, H1), lambda i: (0, 0)),
        pl.BlockSpec((EMB, H1), lambda i: (0, 0)),
        pl.BlockSpec((NFX - 2, H1), lambda i: (0, 0)),
        pl.BlockSpec((1, H1), lambda i: (0, 0)),
        pl.BlockSpec((H1, H2), lambda i: (0, 0)),
        pl.BlockSpec((1, H2), lambda i: (0, 0)),
        pl.BlockSpec((1, H2), lambda i: (0, 0)),
        pl.BlockSpec((1, 1), lambda i: (0, 0)),
    ],
    out_specs=pl.BlockSpec((BLK, 1), lambda i: (i, 0)),
    out_shape=jax.ShapeDtypeStruct((B, 1), jnp.float32),
    compiler_params=pltpu.CompilerParams(dimension_semantics=("arbitrary",)),
)


def kernel(x, users, animes, W1, b1, W2, b2, W3, b3):
    uidx = x[:, 0].astype(jnp.int32).reshape(B // CH, CH)
    aidx = x[:, 1].astype(jnp.int32).reshape(B // CH, CH)
    users_p = jnp.pad(users, ((0, 0), (0, PITCH - EMB)))
    animes_p = jnp.pad(animes, ((0, 0), (0, PITCH - EMB)))
    urows, arows = _gather(uidx, aidx, users_p, animes_p)
    w1u = W1[:, :EMB].T
    w1a = W1[:, EMB:2 * EMB].T
    w1f = W1[:, 2 * EMB:].T
    return _mlp(urows, arows, x, w1u, w1a, w1f, b1.reshape(1, H1),
                W2.T, b2.reshape(1, H2), W3, b3.reshape(1, 1))


# speedup vs baseline: 2.2139x; 1.6593x over previous
import jax
import jax.numpy as jnp
from jax import lax
from jax.experimental import pallas as pl
from jax.experimental.pallas import tpu as pltpu
from jax.experimental.pallas import tpu_sc as plsc

B = 16384
EMB = 100
PITCH = 128
NFX = 56
NC, NS = 2, 16
NW = NC * NS
BPW = B // NW
CH = 128
NCH = BPW // CH
H1, H2 = 128, 32
BLK = 2048


def _gather_body(uidx_ref, aidx_ref, users_ref, animes_ref,
                 urows_out, arows_out, idx_v, b0, b1, b2, b3, gsem, osem):
    bufs = (b0, b1, b2, b3)
    wid = lax.axis_index("s") * NC + lax.axis_index("c")
    base = wid * BPW
    pltpu.sync_copy(uidx_ref.at[pl.ds(wid * NCH, NCH)], idx_v.at[pl.ds(0, NCH)])
    pltpu.sync_copy(aidx_ref.at[pl.ds(wid * NCH, NCH)], idx_v.at[pl.ds(NCH, NCH)])
    ug = [pltpu.async_copy(users_ref.at[idx_v.at[k]], bufs[k], gsem)
          for k in range(NCH)]
    uo = []
    for k in range(NCH):
        ug[k].wait()
        uo.append(pltpu.async_copy(
            bufs[k], urows_out.at[pl.ds(base + k * CH, CH)], osem))
    ag, ao = [], []
    for k in range(NCH):
        uo[k].wait()
        ag.append(pltpu.async_copy(
            animes_ref.at[idx_v.at[NCH + k]], bufs[k], gsem))
    for k in range(NCH):
        ag[k].wait()
        ao.append(pltpu.async_copy(
            bufs[k], arows_out.at[pl.ds(base + k * CH, CH)], osem))
    for k in range(NCH):
        ao[k].wait()


_gather = pl.kernel(
    _gather_body,
    out_type=(jax.ShapeDtypeStruct((B, PITCH), jnp.float32),
              jax.ShapeDtypeStruct((B, PITCH), jnp.float32)),
    mesh=plsc.VectorSubcoreMesh(core_axis_name="c", subcore_axis_name="s",
                                num_cores=NC, num_subcores=NS),
    scratch_types=[
        pltpu.VMEM((2 * NCH, CH), jnp.int32),
        pltpu.VMEM((CH, PITCH), jnp.float32),
        pltpu.VMEM((CH, PITCH), jnp.float32),
        pltpu.VMEM((CH, PITCH), jnp.float32),
        pltpu.VMEM((CH, PITCH), jnp.float32),
        pltpu.SemaphoreType.DMA,
        pltpu.SemaphoreType.DMA,
    ],
)


BLKU = 1024


def _repack_body(in_ref, out_ref):
    t = in_ref[:].T
    out_ref[:] = jnp.concatenate(
        [t, jnp.zeros((t.shape[0], PITCH - EMB), jnp.float32)], axis=1)


def _repack(tab_t):
    n = tab_t.shape[1]
    grid = (pl.cdiv(n, BLKU),)
    return pl.pallas_call(
        _repack_body,
        grid=grid,
        in_specs=[pl.BlockSpec((EMB, BLKU), lambda i: (0, i))],
        out_specs=pl.BlockSpec((BLKU, PITCH), lambda i: (i, 0)),
        out_shape=jax.ShapeDtypeStruct((n, PITCH), jnp.float32),
        compiler_params=pltpu.CompilerParams(dimension_semantics=("arbitrary",)),
    )(tab_t)


def _mlp_body(u_ref, a_ref, x_ref, w1u_ref, w1a_ref, w1f_ref, b1_ref,
              w2_ref, b2_ref, w3_ref, b3_ref, o_ref):
    f = x_ref[:, 2:]
    h = (jnp.dot(u_ref[:, :EMB], w1u_ref[:], preferred_element_type=jnp.float32)
         + jnp.dot(a_ref[:, :EMB], w1a_ref[:], preferred_element_type=jnp.float32)
         + jnp.dot(f, w1f_ref[:], preferred_element_type=jnp.float32)
         + b1_ref[:])
    h = jnp.maximum(h, 0.0)
    h2 = jnp.dot(h, w2_ref[:], preferred_element_type=jnp.float32) + b2_ref[:]
    h2 = jnp.maximum(h2, 0.0)
    z = jnp.sum(h2 * w3_ref[:], axis=1, keepdims=True) + b3_ref[:]
    o_ref[:] = jax.nn.sigmoid(z)


_mlp = pl.pallas_call(
    _mlp_body,
    grid=(B // BLK,),
    in_specs=[
        pl.BlockSpec((BLK, PITCH), lambda i: (i, 0)),
        pl.BlockSpec((BLK, PITCH), lambda i: (i, 0)),
        pl.BlockSpec((BLK, NFX), lambda i: (i, 0)),
        pl.BlockSpec((EMB, H1), lambda i: (0, 0)),
        pl.BlockSpec((EMB, H1), lambda i: (0, 0)),
        pl.BlockSpec((NFX - 2, H1), lambda i: (0, 0)),
        pl.BlockSpec((1, H1), lambda i: (0, 0)),
        pl.BlockSpec((H1, H2), lambda i: (0, 0)),
        pl.BlockSpec((1, H2), lambda i: (0, 0)),
        pl.BlockSpec((1, H2), lambda i: (0, 0)),
        pl.BlockSpec((1, 1), lambda i: (0, 0)),
    ],
    out_specs=pl.BlockSpec((BLK, 1), lambda i: (i, 0)),
    out_shape=jax.ShapeDtypeStruct((B, 1), jnp.float32),
    compiler_params=pltpu.CompilerParams(dimension_semantics=("arbitrary",)),
)


def kernel(x, users, animes, W1, b1, W2, b2, W3, b3):
    uidx = x[:, 0].astype(jnp.int32).reshape(B // CH, CH)
    aidx = x[:, 1].astype(jnp.int32).reshape(B // CH, CH)
    users_p = _repack(users.T)
    animes_p = _repack(animes.T)
    urows, arows = _gather(uidx, aidx, users_p, animes_p)
    w1u = W1[:, :EMB].T
    w1a = W1[:, EMB:2 * EMB].T
    w1f = W1[:, 2 * EMB:].T
    return _mlp(urows, arows, x, w1u, w1a, w1f, b1.reshape(1, H1),
                W2.T, b2.reshape(1, H2), W3, b3.reshape(1, 1))
